# baseline (speedup 1.0000x reference)
import functools
import math

import jax
import jax.numpy as jnp
from jax import lax
from jax.experimental import pallas as pl
from jax.experimental.pallas import tpu as pltpu
from jax.experimental.pallas import tpu_sc as plsc

SMOOTHING = 0.1
CONFIDENCE = 1.0 - SMOOTHING

_NC = 2
_NS = 16
_NW = _NC * _NS


def _sc_gather(x_flat, flat_idx, n):
    bpw = n // _NW
    mesh = plsc.VectorSubcoreMesh(core_axis_name="c", subcore_axis_name="s")

    @functools.partial(
        pl.kernel,
        out_type=jax.ShapeDtypeStruct((n,), jnp.float32),
        mesh=mesh,
        scratch_types=[
            pltpu.VMEM((bpw,), jnp.int32),
            pltpu.VMEM((bpw,), jnp.float32),
            pltpu.SemaphoreType.DMA,
        ],
    )
    def gather_kernel(x_hbm, idx_hbm, out_hbm, idx_v, vals_v, sem):
        wid = lax.axis_index("s") * _NC + lax.axis_index("c")
        base = wid * bpw
        pltpu.sync_copy(idx_hbm.at[pl.ds(base, bpw)], idx_v)
        pltpu.async_copy(x_hbm.at[idx_v], vals_v, sem).wait()
        pltpu.sync_copy(vals_v, out_hbm.at[pl.ds(base, bpw)])

    return gather_kernel(x_flat, flat_idx)


def _tc_reduce(x, g2d, const_term, eps):
    n, size = x.shape
    rows = 32
    grid = n // rows
    cm = eps - CONFIDENCE

    def body(x_ref, g_ref, o_ref):
        i = pl.program_id(0)

        @pl.when(i == 0)
        def _():
            o_ref[0, 0] = jnp.float32(const_term) + jnp.float32(cm) * jnp.sum(
                jnp.log(g_ref[...]))

        o_ref[0, 0] += jnp.float32(-eps) * jnp.sum(jnp.log(x_ref[...]))

    return pl.pallas_call(
        body,
        grid=(grid,),
        in_specs=[
            pl.BlockSpec((rows, size), lambda i: (i, 0)),
            pl.BlockSpec(g2d.shape, lambda i: (0, 0)),
        ],
        out_specs=pl.BlockSpec(memory_space=pltpu.SMEM),
        out_shape=jax.ShapeDtypeStruct((1, 1), jnp.float32),
        compiler_params=pltpu.CompilerParams(
            dimension_semantics=("arbitrary",),
        ),
    )(x, g2d)


def kernel(x, target):
    n, size = x.shape
    eps = SMOOTHING / (size - 1)
    const_term = n * ((size - 1) * eps * math.log(eps)
                      + CONFIDENCE * math.log(CONFIDENCE))

    flat_idx = jnp.arange(n, dtype=jnp.int32) * size + target
    g = _sc_gather(x.reshape(-1), flat_idx, n)
    out = _tc_reduce(x, g.reshape(8, n // 8), const_term, eps)
    return out[0, 0]


# speedup vs baseline: 2.0484x; 2.0484x over previous
import math

import jax
import jax.numpy as jnp
from jax.experimental import pallas as pl
from jax.experimental.pallas import tpu as pltpu

SMOOTHING = 0.1
CONFIDENCE = 1.0 - SMOOTHING
_ROWS = 32


def _body(x_ref, t_ref, o_ref, *, const_term, eps):
    i = pl.program_id(0)

    @pl.when(i == 0)
    def _():
        o_ref[0, 0] = jnp.float32(const_term)

    logx = jnp.log(x_ref[...])
    col = jax.lax.broadcasted_iota(jnp.int32, x_ref.shape, 1)
    tgt = jnp.sum(jnp.where(col == t_ref[...], logx, 0.0))
    o_ref[0, 0] += jnp.float32(-eps) * jnp.sum(logx) + jnp.float32(
        eps - CONFIDENCE) * tgt


def kernel(x, target):
    n, size = x.shape
    eps = SMOOTHING / (size - 1)
    const_term = n * ((size - 1) * eps * math.log(eps)
                      + CONFIDENCE * math.log(CONFIDENCE))

    import functools
    body = functools.partial(_body, const_term=const_term, eps=eps)
    out = pl.pallas_call(
        body,
        grid=(n // _ROWS,),
        in_specs=[
            pl.BlockSpec((_ROWS, size), lambda i: (i, 0)),
            pl.BlockSpec((_ROWS, 1), lambda i: (i, 0)),
        ],
        out_specs=pl.BlockSpec(memory_space=pltpu.SMEM),
        out_shape=jax.ShapeDtypeStruct((1, 1), jnp.float32),
        compiler_params=pltpu.CompilerParams(
            dimension_semantics=("arbitrary",),
        ),
    )(x, target.reshape(n, 1))
    return out[0, 0]


# speedup vs baseline: 5.8978x; 2.8793x over previous
import functools
import math

import jax
import jax.numpy as jnp
from jax.experimental import pallas as pl
from jax.experimental.pallas import tpu as pltpu

SMOOTHING = 0.1
CONFIDENCE = 1.0 - SMOOTHING
_BLK = 1000


def _body(xt_ref, t_ref, o_ref, *, const_term, eps, blk):
    i = pl.program_id(0)

    @pl.when(i == 0)
    def _():
        o_ref[0, 0] = jnp.float32(const_term)

    logx = jnp.log(xt_ref[...])
    row = i * blk + jax.lax.broadcasted_iota(jnp.int32, xt_ref.shape, 0)
    tgt = jnp.sum(jnp.where(row == t_ref[...], logx, 0.0))
    o_ref[0, 0] += jnp.float32(-eps) * jnp.sum(logx) + jnp.float32(
        eps - CONFIDENCE) * tgt


def kernel(x, target):
    n, size = x.shape
    eps = SMOOTHING / (size - 1)
    const_term = n * ((size - 1) * eps * math.log(eps)
                      + CONFIDENCE * math.log(CONFIDENCE))

    xt = x.T
    body = functools.partial(_body, const_term=const_term, eps=eps, blk=_BLK)
    out = pl.pallas_call(
        body,
        grid=(size // _BLK,),
        in_specs=[
            pl.BlockSpec((_BLK, n), lambda i: (i, 0)),
            pl.BlockSpec((1, n), lambda i: (0, 0)),
        ],
        out_specs=pl.BlockSpec(memory_space=pltpu.SMEM),
        out_shape=jax.ShapeDtypeStruct((1, 1), jnp.float32),
        compiler_params=pltpu.CompilerParams(
            dimension_semantics=("arbitrary",),
        ),
    )(xt, target.reshape(1, n))
    return out[0, 0]
